# baseline (device time: 37251 ns/iter reference)
import jax
import jax.numpy as jnp
from jax import lax
from jax.experimental import pallas as pl
from jax.experimental.pallas import tpu as pltpu

N_LAYERS = 3
C = 8


def kernel(x, Win0, Wout0, Win1, Wout1, Win2, Wout2):
    b, d_y = x.shape
    _, h_x = Win0.shape
    bc = b // C

    def body(x_hbm, win0_hbm, wout0_hbm, win1_hbm, wout1_hbm, win2_hbm,
             wout2_hbm, out_hbm,
             xv_ref, wv_ref, wov_ref, outv_ref,
             ysend_ref, yrecv_ref, xsend_ref, xrecv_ref,
             in_sems, out_sems,
             ysend_sems, yrecv_sems, xsend_sems, xrecv_sems):
        my_x = lax.axis_index("x")
        my_y = lax.axis_index("y")
        y_nbr = (my_x, 1 - my_y)
        x_nbr = (1 - my_x, my_y)

        cp_x = pltpu.make_async_copy(x_hbm, xv_ref, in_sems.at[0])
        cp_x.start()
        win_cps = []
        wout_cps = []
        for l, (w_hbm, wo_hbm) in enumerate(
                [(win0_hbm, wout0_hbm), (win1_hbm, wout1_hbm),
                 (win2_hbm, wout2_hbm)]):
            cp = pltpu.make_async_copy(w_hbm, wv_ref.at[l],
                                       in_sems.at[1 + 2 * l])
            cp.start()
            win_cps.append(cp)
            cp = pltpu.make_async_copy(wo_hbm, wov_ref.at[l],
                                       in_sems.at[2 + 2 * l])
            cp.start()
            wout_cps.append(cp)

        barrier_sem = pltpu.get_barrier_semaphore()
        for nbr in (y_nbr, x_nbr):
            pl.semaphore_signal(barrier_sem, inc=1, device_id=nbr,
                                device_id_type=pl.DeviceIdType.MESH)
        pl.semaphore_wait(barrier_sem, 2)

        y_rd = [None] * C
        x_rd = [None] * C
        out_cps = []
        q = [None] * C

        cp_x.wait()
        for l in range(N_LAYERS):
            win_cps[l].wait()
            wb = wv_ref[l].astype(jnp.bfloat16)
            p = [None] * C
            for c in range(C):
                if l == 0:
                    xb_c = xv_ref[pl.ds(c * bc, bc), :].astype(jnp.bfloat16)
                else:
                    x_rd[c].wait_recv()
                    x_rd[c].wait_send()
                    xb_c = (q[c] + xrecv_ref[l - 1, c].astype(jnp.float32)
                            ).astype(jnp.bfloat16)
                p[c] = jnp.dot(xb_c, wb, preferred_element_type=jnp.float32)
                ysend_ref[c] = p[c].astype(jnp.bfloat16)
                rd = pltpu.make_async_remote_copy(
                    src_ref=ysend_ref.at[c], dst_ref=yrecv_ref.at[l, c],
                    send_sem=ysend_sems.at[c], recv_sem=yrecv_sems.at[l, c],
                    device_id=y_nbr, device_id_type=pl.DeviceIdType.MESH)
                rd.start()
                y_rd[c] = rd

            wout_cps[l].wait()
            wob = wov_ref[l].astype(jnp.bfloat16)
            for c in range(C):
                y_rd[c].wait_recv()
                y_rd[c].wait_send()
                h = jnp.maximum(p[c] + yrecv_ref[l, c].astype(jnp.float32),
                                0.0).astype(jnp.bfloat16)
                q[c] = jnp.dot(h, wob, preferred_element_type=jnp.float32)
                xsend_ref[c] = q[c].astype(jnp.bfloat16)
                rd = pltpu.make_async_remote_copy(
                    src_ref=xsend_ref.at[c], dst_ref=xrecv_ref.at[l, c],
                    send_sem=xsend_sems.at[c], recv_sem=xrecv_sems.at[l, c],
                    device_id=x_nbr, device_id_type=pl.DeviceIdType.MESH)
                rd.start()
                x_rd[c] = rd

        for c in range(C):
            x_rd[c].wait_recv()
            x_rd[c].wait_send()
            outv_ref[pl.ds(c * bc, bc), :] = (
                q[c] + xrecv_ref[N_LAYERS - 1, c].astype(jnp.float32))
            cp = pltpu.make_async_copy(
                outv_ref.at[pl.ds(c * bc, bc), :],
                out_hbm.at[pl.ds(c * bc, bc), :],
                out_sems.at[c])
            cp.start()
            out_cps.append(cp)
        for cp in out_cps:
            cp.wait()

    return pl.pallas_call(
        body,
        out_shape=jax.ShapeDtypeStruct((b, d_y), jnp.float32),
        in_specs=[pl.BlockSpec(memory_space=pltpu.MemorySpace.HBM)] * 7,
        out_specs=pl.BlockSpec(memory_space=pltpu.MemorySpace.HBM),
        scratch_shapes=[
            pltpu.VMEM((b, d_y), jnp.float32),
            pltpu.VMEM((N_LAYERS, d_y, h_x), jnp.float32),
            pltpu.VMEM((N_LAYERS, h_x, d_y), jnp.float32),
            pltpu.VMEM((b, d_y), jnp.float32),
            pltpu.VMEM((C, bc, h_x), jnp.bfloat16),
            pltpu.VMEM((N_LAYERS, C, bc, h_x), jnp.bfloat16),
            pltpu.VMEM((C, bc, d_y), jnp.bfloat16),
            pltpu.VMEM((N_LAYERS, C, bc, d_y), jnp.bfloat16),
            pltpu.SemaphoreType.DMA((7,)),
            pltpu.SemaphoreType.DMA((C,)),
            pltpu.SemaphoreType.DMA((C,)),
            pltpu.SemaphoreType.DMA((N_LAYERS, C)),
            pltpu.SemaphoreType.DMA((C,)),
            pltpu.SemaphoreType.DMA((N_LAYERS, C)),
        ],
        compiler_params=pltpu.CompilerParams(collective_id=0),
    )(x, Win0, Wout0, Win1, Wout1, Win2, Wout2)


# device time: 33958 ns/iter; 1.0970x vs baseline; 1.0970x over previous
import jax
import jax.numpy as jnp
from jax import lax
from jax.experimental import pallas as pl
from jax.experimental.pallas import tpu as pltpu

N_LAYERS = 3
C = 8


def kernel(x, Win0, Wout0, Win1, Wout1, Win2, Wout2):
    b, d_y = x.shape
    _, h_x = Win0.shape
    bc = b // C

    xb = x.astype(jnp.bfloat16)
    win_s = jnp.stack([Win0, Win1, Win2]).astype(jnp.bfloat16)
    wout_s = jnp.stack([Wout0, Wout1, Wout2]).astype(jnp.bfloat16)

    def body(x_ref, win_ref, wout_ref, out_ref,
             ysend_ref, yrecv_ref, xsend_ref, xrecv_ref,
             ysend_sems, yrecv_sems, xsend_sems, xrecv_sems):
        my_x = lax.axis_index("x")
        my_y = lax.axis_index("y")
        y_nbr = (my_x, 1 - my_y)
        x_nbr = (1 - my_x, my_y)

        barrier_sem = pltpu.get_barrier_semaphore()
        for nbr in (y_nbr, x_nbr):
            pl.semaphore_signal(barrier_sem, inc=1, device_id=nbr,
                                device_id_type=pl.DeviceIdType.MESH)
        pl.semaphore_wait(barrier_sem, 2)

        y_rd = [None] * C
        x_rd = [None] * C
        q = [None] * C

        for l in range(N_LAYERS):
            wb = win_ref[l]
            p = [None] * C
            for c in range(C):
                if l == 0:
                    xb_c = x_ref[pl.ds(c * bc, bc), :]
                else:
                    x_rd[c].wait_recv()
                    x_rd[c].wait_send()
                    xb_c = (q[c] + xrecv_ref[l - 1, c].astype(jnp.float32)
                            ).astype(jnp.bfloat16)
                p[c] = jnp.dot(xb_c, wb, preferred_element_type=jnp.float32)
                ysend_ref[c] = p[c].astype(jnp.bfloat16)
                rd = pltpu.make_async_remote_copy(
                    src_ref=ysend_ref.at[c], dst_ref=yrecv_ref.at[l, c],
                    send_sem=ysend_sems.at[c], recv_sem=yrecv_sems.at[l, c],
                    device_id=y_nbr, device_id_type=pl.DeviceIdType.MESH)
                rd.start()
                y_rd[c] = rd

            wob = wout_ref[l]
            for c in range(C):
                y_rd[c].wait_recv()
                y_rd[c].wait_send()
                h = jnp.maximum(p[c] + yrecv_ref[l, c].astype(jnp.float32),
                                0.0).astype(jnp.bfloat16)
                q[c] = jnp.dot(h, wob, preferred_element_type=jnp.float32)
                xsend_ref[c] = q[c].astype(jnp.bfloat16)
                rd = pltpu.make_async_remote_copy(
                    src_ref=xsend_ref.at[c], dst_ref=xrecv_ref.at[l, c],
                    send_sem=xsend_sems.at[c], recv_sem=xrecv_sems.at[l, c],
                    device_id=x_nbr, device_id_type=pl.DeviceIdType.MESH)
                rd.start()
                x_rd[c] = rd

        for c in range(C):
            x_rd[c].wait_recv()
            x_rd[c].wait_send()
            out_ref[pl.ds(c * bc, bc), :] = (
                q[c] + xrecv_ref[N_LAYERS - 1, c].astype(jnp.float32)
            ).astype(jnp.bfloat16)

    return pl.pallas_call(
        body,
        out_shape=jax.ShapeDtypeStruct((b, d_y), jnp.bfloat16),
        in_specs=[pl.BlockSpec(memory_space=pltpu.VMEM)] * 3,
        out_specs=pl.BlockSpec(memory_space=pltpu.VMEM),
        scratch_shapes=[
            pltpu.VMEM((C, bc, h_x), jnp.bfloat16),
            pltpu.VMEM((N_LAYERS, C, bc, h_x), jnp.bfloat16),
            pltpu.VMEM((C, bc, d_y), jnp.bfloat16),
            pltpu.VMEM((N_LAYERS, C, bc, d_y), jnp.bfloat16),
            pltpu.SemaphoreType.DMA((C,)),
            pltpu.SemaphoreType.DMA((N_LAYERS, C)),
            pltpu.SemaphoreType.DMA((C,)),
            pltpu.SemaphoreType.DMA((N_LAYERS, C)),
        ],
        compiler_params=pltpu.CompilerParams(collective_id=0),
    )(xb, win_s, wout_s)


# device time: 31084 ns/iter; 1.1984x vs baseline; 1.0925x over previous
import jax
import jax.numpy as jnp
from jax import lax
from jax.experimental import pallas as pl
from jax.experimental.pallas import tpu as pltpu

N_LAYERS = 3
C = 8


def kernel(x, Win0, Wout0, Win1, Wout1, Win2, Wout2):
    b, d_y = x.shape
    _, h_x = Win0.shape
    bc = b // C

    xb = x.astype(jnp.bfloat16)
    win_s = jnp.stack([Win0, Win1, Win2]).astype(jnp.bfloat16)
    wout_s = jnp.stack([Wout0, Wout1, Wout2]).astype(jnp.bfloat16)

    def body(x_ref, win_ref, wout_ref, out_ref,
             ysend_ref, yrecv_ref, xsend_ref, xrecv_ref,
             ysend_sems, yrecv_sems, xsend_sems, xrecv_sems):
        my_x = lax.axis_index("x")
        my_y = lax.axis_index("y")
        y_nbr = (my_x, 1 - my_y)
        x_nbr = (1 - my_x, my_y)

        barrier_sem = pltpu.get_barrier_semaphore()
        for nbr in (y_nbr, x_nbr):
            pl.semaphore_signal(barrier_sem, inc=1, device_id=nbr,
                                device_id_type=pl.DeviceIdType.MESH)
        pl.semaphore_wait(barrier_sem, 2)

        y_rd = [None] * C
        x_rd = [None] * C


        def mm1_ysend(l, c):
            if l == 0:
                xb_c = x_ref[pl.ds(c * bc, bc), :]
            else:
                x_rd[c].wait_recv()
                x_rd[c].wait_send()
                xb_c = xsend_ref[c] + xrecv_ref[l - 1, c]
            p_c = jnp.dot(xb_c, win_ref[l],
                          preferred_element_type=jnp.float32
                          ).astype(jnp.bfloat16)
            ysend_ref[c] = p_c
            rd = pltpu.make_async_remote_copy(
                src_ref=ysend_ref.at[c], dst_ref=yrecv_ref.at[l, c],
                send_sem=ysend_sems.at[c], recv_sem=yrecv_sems.at[l, c],
                device_id=y_nbr, device_id_type=pl.DeviceIdType.MESH)
            rd.start()
            y_rd[c] = rd

        def mm2_xsend(l, c):
            y_rd[c].wait_recv()
            y_rd[c].wait_send()
            h = jnp.maximum(ysend_ref[c] + yrecv_ref[l, c], 0.0)
            q_c = jnp.dot(h, wout_ref[l],
                          preferred_element_type=jnp.float32
                          ).astype(jnp.bfloat16)
            xsend_ref[c] = q_c
            rd = pltpu.make_async_remote_copy(
                src_ref=xsend_ref.at[c], dst_ref=xrecv_ref.at[l, c],
                send_sem=xsend_sems.at[c], recv_sem=xrecv_sems.at[l, c],
                device_id=x_nbr, device_id_type=pl.DeviceIdType.MESH)
            rd.start()
            x_rd[c] = rd

        def store_out(c):
            x_rd[c].wait_recv()
            x_rd[c].wait_send()
            out_ref[pl.ds(c * bc, bc), :] = (
                xsend_ref[c] + xrecv_ref[N_LAYERS - 1, c])

        LAG = 3
        for c in range(C):
            mm1_ysend(0, c)
        for l in range(N_LAYERS):
            last = l + 1 == N_LAYERS
            for c in range(C):
                mm2_xsend(l, c)
                if c >= LAG:
                    store_out(c - LAG) if last else mm1_ysend(l + 1, c - LAG)
            for c in range(C - LAG, C):
                store_out(c) if last else mm1_ysend(l + 1, c)

    return pl.pallas_call(
        body,
        out_shape=jax.ShapeDtypeStruct((b, d_y), jnp.bfloat16),
        in_specs=[pl.BlockSpec(memory_space=pltpu.VMEM)] * 3,
        out_specs=pl.BlockSpec(memory_space=pltpu.VMEM),
        scratch_shapes=[
            pltpu.VMEM((C, bc, h_x), jnp.bfloat16),
            pltpu.VMEM((N_LAYERS, C, bc, h_x), jnp.bfloat16),
            pltpu.VMEM((C, bc, d_y), jnp.bfloat16),
            pltpu.VMEM((N_LAYERS, C, bc, d_y), jnp.bfloat16),
            pltpu.SemaphoreType.DMA((C,)),
            pltpu.SemaphoreType.DMA((N_LAYERS, C)),
            pltpu.SemaphoreType.DMA((C,)),
            pltpu.SemaphoreType.DMA((N_LAYERS, C)),
        ],
        compiler_params=pltpu.CompilerParams(collective_id=0),
    )(xb, win_s, wout_s)


# device time: 31034 ns/iter; 1.2003x vs baseline; 1.0016x over previous
import jax
import jax.numpy as jnp
from jax import lax
from jax.experimental import pallas as pl
from jax.experimental.pallas import tpu as pltpu

N_LAYERS = 3
C = 8


def kernel(x, Win0, Wout0, Win1, Wout1, Win2, Wout2):
    b, d_y = x.shape
    _, h_x = Win0.shape
    bc = b // C

    xb = x.astype(jnp.bfloat16)
    win_s = jnp.stack([Win0, Win1, Win2]).astype(jnp.bfloat16)
    wout_s = jnp.stack([Wout0, Wout1, Wout2]).astype(jnp.bfloat16)

    def body(x_ref, win_ref, wout_ref, out_ref,
             ysend_ref, yrecv_ref, xsend_ref, xrecv_ref,
             ysend_sems, yrecv_sems, xsend_sems, xrecv_sems):
        my_x = lax.axis_index("x")
        my_y = lax.axis_index("y")
        y_nbr = (my_x, 1 - my_y)
        x_nbr = (1 - my_x, my_y)

        p0 = jnp.dot(x_ref[pl.ds(0, bc), :], win_ref[0],
                     preferred_element_type=jnp.float32).astype(jnp.bfloat16)
        ysend_ref[0] = p0
        barrier_sem = pltpu.get_barrier_semaphore()
        for nbr in (y_nbr, x_nbr):
            pl.semaphore_signal(barrier_sem, inc=1, device_id=nbr,
                                device_id_type=pl.DeviceIdType.MESH)
        pl.semaphore_wait(barrier_sem, 2)

        y_rd = [None] * C
        x_rd = [None] * C


        def mm1_ysend(l, c):
            if l == 0 and c == 0:
                xb_c = None
            elif l == 0:
                xb_c = x_ref[pl.ds(c * bc, bc), :]
            else:
                x_rd[c].wait_recv()
                x_rd[c].wait_send()
                xb_c = xsend_ref[c] + xrecv_ref[l - 1, c]
            if xb_c is not None:
                p_c = jnp.dot(xb_c, win_ref[l],
                              preferred_element_type=jnp.float32
                              ).astype(jnp.bfloat16)
                ysend_ref[c] = p_c
            rd = pltpu.make_async_remote_copy(
                src_ref=ysend_ref.at[c], dst_ref=yrecv_ref.at[l, c],
                send_sem=ysend_sems.at[c], recv_sem=yrecv_sems.at[l, c],
                device_id=y_nbr, device_id_type=pl.DeviceIdType.MESH)
            rd.start()
            y_rd[c] = rd

        def mm2_xsend(l, c):
            y_rd[c].wait_recv()
            y_rd[c].wait_send()
            h = jnp.maximum(ysend_ref[c] + yrecv_ref[l, c], 0.0)
            q_c = jnp.dot(h, wout_ref[l],
                          preferred_element_type=jnp.float32
                          ).astype(jnp.bfloat16)
            xsend_ref[c] = q_c
            rd = pltpu.make_async_remote_copy(
                src_ref=xsend_ref.at[c], dst_ref=xrecv_ref.at[l, c],
                send_sem=xsend_sems.at[c], recv_sem=xrecv_sems.at[l, c],
                device_id=x_nbr, device_id_type=pl.DeviceIdType.MESH)
            rd.start()
            x_rd[c] = rd

        def store_out(c):
            x_rd[c].wait_recv()
            x_rd[c].wait_send()
            out_ref[pl.ds(c * bc, bc), :] = (
                xsend_ref[c] + xrecv_ref[N_LAYERS - 1, c])

        LAG = 3
        for c in range(C):
            mm1_ysend(0, c)
        for l in range(N_LAYERS):
            last = l + 1 == N_LAYERS
            for c in range(C):
                mm2_xsend(l, c)
                if c >= LAG:
                    store_out(c - LAG) if last else mm1_ysend(l + 1, c - LAG)
            for c in range(C - LAG, C):
                store_out(c) if last else mm1_ysend(l + 1, c)

    return pl.pallas_call(
        body,
        out_shape=jax.ShapeDtypeStruct((b, d_y), jnp.bfloat16),
        in_specs=[pl.BlockSpec(memory_space=pltpu.VMEM)] * 3,
        out_specs=pl.BlockSpec(memory_space=pltpu.VMEM),
        scratch_shapes=[
            pltpu.VMEM((C, bc, h_x), jnp.bfloat16),
            pltpu.VMEM((N_LAYERS, C, bc, h_x), jnp.bfloat16),
            pltpu.VMEM((C, bc, d_y), jnp.bfloat16),
            pltpu.VMEM((N_LAYERS, C, bc, d_y), jnp.bfloat16),
            pltpu.SemaphoreType.DMA((C,)),
            pltpu.SemaphoreType.DMA((N_LAYERS, C)),
            pltpu.SemaphoreType.DMA((C,)),
            pltpu.SemaphoreType.DMA((N_LAYERS, C)),
        ],
        compiler_params=pltpu.CompilerParams(collective_id=0),
    )(xb, win_s, wout_s)
